# baseline (speedup 1.0000x reference)
import jax
import jax.numpy as jnp
from jax import lax
from jax.experimental import pallas as pl
from jax.experimental.pallas import tpu as pltpu
from jax.experimental.pallas import tpu_sc as plsc

B = 16384
D = 64
NC = 2
NS = 16
NW = NC * NS
BPW = B // NW
CHUNK = 128
NCH = BPW // CHUNK
GROUPS = BPW // 16


def _body(users_hbm, items_hbm, ut_hbm, it_hbm, out_hbm,
          idx_u, idx_i, rows_u, rows_i, out_v, sem_u, sem_i):
    c = lax.axis_index("c")
    s = lax.axis_index("s")
    wid = s * NC + c
    base = wid * BPW

    for ch in range(NCH):
        pltpu.sync_copy(users_hbm.at[pl.ds(base + ch * CHUNK, CHUNK)],
                        idx_u.at[ch])
        pltpu.sync_copy(items_hbm.at[pl.ds(base + ch * CHUNK, CHUNK)],
                        idx_i.at[ch])

    copies = []
    for ch in range(NCH):
        copies.append(pltpu.async_copy(
            ut_hbm.at[idx_u.at[ch]],
            rows_u.at[pl.ds(ch * CHUNK, CHUNK)], sem_u))
        copies.append(pltpu.async_copy(
            it_hbm.at[idx_i.at[ch]],
            rows_i.at[pl.ds(ch * CHUNK, CHUNK)], sem_i))
    for cp in copies:
        cp.wait()

    lanes = lax.iota(jnp.int32, 16)

    def group_body(g, _):
        rows = g * 16 + lanes
        acc = jnp.zeros((16,), jnp.float32)
        for d in range(D):
            col = jnp.full((16,), d, jnp.int32)
            u = plsc.load_gather(rows_u, [rows, col])
            v = plsc.load_gather(rows_i, [rows, col])
            acc = acc + u * v
        out_v[pl.ds(g * 16, 16)] = acc
        return 0

    lax.fori_loop(0, GROUPS, group_body, 0)

    pltpu.sync_copy(out_v, out_hbm.at[pl.ds(base, BPW)])


def kernel(users, items, user_table, item_table):
    mesh = plsc.VectorSubcoreMesh(core_axis_name="c", subcore_axis_name="s")
    fn = pl.kernel(
        _body,
        out_type=jax.ShapeDtypeStruct((B,), jnp.float32),
        mesh=mesh,
        compiler_params=pltpu.CompilerParams(
            needs_layout_passes=False, use_tc_tiling_on_sc=False),
        scratch_types=[
            pltpu.VMEM((NCH, CHUNK), jnp.int32),
            pltpu.VMEM((NCH, CHUNK), jnp.int32),
            pltpu.VMEM((BPW, D), jnp.float32),
            pltpu.VMEM((BPW, D), jnp.float32),
            pltpu.VMEM((BPW,), jnp.float32),
            pltpu.SemaphoreType.DMA,
            pltpu.SemaphoreType.DMA,
        ],
    )
    return fn(users.astype(jnp.int32), items.astype(jnp.int32),
              user_table, item_table)
